# baseline (speedup 1.0000x reference)
import functools

import jax
import jax.numpy as jnp
from jax import lax
from jax.experimental import pallas as pl
from jax.experimental.pallas import tpu as pltpu
from jax.experimental.pallas import tpu_sc as plsc

_N = 10000
_E = 160000
_D = 256
_G = 64
_NC = 2
_NS = 16
_LANE = 128
_EPW = 10240
_EPAD = _NS * _EPW
_NCHUNK = _EPW // _LANE
_ACCROWS = 10240
_DUMMY = _N
_ZROWS = _ACCROWS // _NS
_OROWS = _ACCROWS // _NS
_R = 400
_NB = _N // _R
_EPS = 1e-5


def _sc_degree(dsts, zrows16, ones16):
    mesh = plsc.VectorSubcoreMesh(core_axis_name="c", subcore_axis_name="s")

    @functools.partial(
        pl.kernel,
        out_type=jax.ShapeDtypeStruct((_NC, _ACCROWS, 16), jnp.float32),
        mesh=mesh,
        scratch_types=[
            pltpu.VMEM((_NCHUNK, _LANE), jnp.int32),
            pltpu.VMEM((_LANE, 16), jnp.float32),
            pltpu.VMEM_SHARED((_ACCROWS, 16), jnp.float32),
        ],
    )
    def k(dst_hbm, z_hbm, o_hbm, out_hbm, dst_v, ones_v, acc):
        c = lax.axis_index("c")
        s = lax.axis_index("s")
        pltpu.sync_copy(dst_hbm.at[s], dst_v)
        pltpu.sync_copy(o_hbm, ones_v)
        pltpu.sync_copy(z_hbm, acc.at[pl.ds(s * _ZROWS, _ZROWS)])
        plsc.subcore_barrier()

        def chunk(j, carry):
            pltpu.sync_copy(ones_v, acc.at[dst_v.at[j]], add=True)
            return carry

        lax.fori_loop(0, _NCHUNK, chunk, 0)
        plsc.subcore_barrier()
        pltpu.sync_copy(acc.at[pl.ds(s * _OROWS, _OROWS)],
                        out_hbm.at[c, pl.ds(s * _OROWS, _OROWS)])

    return k(dsts, zrows16, ones16)


def _sc_segment_sum(u2, gidx, dsts, zrows):
    mesh = plsc.VectorSubcoreMesh(core_axis_name="c", subcore_axis_name="s")

    @functools.partial(
        pl.kernel,
        out_type=jax.ShapeDtypeStruct((_NC, _ACCROWS, _LANE), jnp.float32),
        mesh=mesh,
        scratch_types=[
            pltpu.VMEM((_NCHUNK, _LANE), jnp.int32),
            pltpu.VMEM((_NCHUNK, _LANE), jnp.int32),
            pltpu.VMEM((_LANE, _LANE), jnp.float32),
            pltpu.VMEM_SHARED((_ACCROWS, _LANE), jnp.float32),
            pltpu.SemaphoreType.DMA,
        ],
    )
    def k(u_hbm, gidx_hbm, dst_hbm, z_hbm, out_hbm, idx_v, dst_v, rows_v,
          acc, sem):
        c = lax.axis_index("c")
        s = lax.axis_index("s")
        pltpu.sync_copy(gidx_hbm.at[c, s], idx_v)
        pltpu.sync_copy(dst_hbm.at[s], dst_v)
        pltpu.sync_copy(z_hbm, acc.at[pl.ds(s * _ZROWS, _ZROWS)])
        plsc.subcore_barrier()

        def chunk(j, carry):
            pltpu.async_copy(u_hbm.at[idx_v.at[j]], rows_v, sem).wait()
            pltpu.sync_copy(rows_v, acc.at[dst_v.at[j]], add=True)
            return carry

        lax.fori_loop(0, _NCHUNK, chunk, 0)
        plsc.subcore_barrier()
        pltpu.sync_copy(acc.at[pl.ds(s * _OROWS, _OROWS)],
                        out_hbm.at[c, pl.ds(s * _OROWS, _OROWS)])

    return k(u2, gidx, dsts, zrows)


def _tc_layer1(x, W, deg_col):
    def body(x_ref, w_ref, deg_ref, hw_ref, u_ref, dinv_ref):
        hw = jnp.dot(x_ref[...], w_ref[...],
                     preferred_element_type=jnp.float32)
        dinv = lax.rsqrt(deg_ref[...] + 1.0)
        hw_ref[...] = hw
        u_ref[...] = hw * dinv
        dinv_ref[...] = dinv

    return pl.pallas_call(
        body,
        grid=(_NB,),
        in_specs=[
            pl.BlockSpec((_R, _D), lambda i: (i, 0)),
            pl.BlockSpec((_D, _D), lambda i: (0, 0)),
            pl.BlockSpec((_R, 1), lambda i: (i, 0)),
        ],
        out_specs=[
            pl.BlockSpec((_R, _D), lambda i: (i, 0)),
            pl.BlockSpec((_R, _D), lambda i: (i, 0)),
            pl.BlockSpec((_R, 1), lambda i: (i, 0)),
        ],
        out_shape=[
            jax.ShapeDtypeStruct((_N, _D), jnp.float32),
            jax.ShapeDtypeStruct((_N, _D), jnp.float32),
            jax.ShapeDtypeStruct((_N, 1), jnp.float32),
        ],
    )(x, W, deg_col)


def _pre_block(s_ref, hw_ref, dinv, b_ref):
    sb = jnp.concatenate([s_ref[0], s_ref[1]], axis=1)
    return dinv * sb + (dinv * dinv) * hw_ref[...] + b_ref[...][None, :]


def _tc_main(sagg, hw, dinv, b, g, be, batch_col, W_next):
    has_next = W_next is not None

    def body(s_ref, hw_ref, dinv_ref, b_ref, g_ref, be_ref, bc_ref,
             *rest):
        if has_next:
            w_ref, hwn_ref, un_ref, psum_ref, pmax_ref, cnt_ref, st_ref = rest
        else:
            psum_ref, pmax_ref, cnt_ref, st_ref = rest
        p = pl.program_id(0)
        i = pl.program_id(1)
        dinv = dinv_ref[...]
        pre = _pre_block(s_ref, hw_ref, dinv, b_ref)

        @pl.when(p == 0)
        def _():
            @pl.when(i == 0)
            def _():
                st_ref[...] = jnp.zeros_like(st_ref)

            st_ref[...] += jnp.concatenate(
                [jnp.sum(pre, axis=0)[None, :],
                 jnp.sum(pre * pre, axis=0)[None, :]], axis=0)

        @pl.when(p == 1)
        def _():
            m = st_ref[0, :] * (1.0 / _N)
            v = st_ref[1, :] * (1.0 / _N) - m * m
            h = jnp.maximum(
                (pre - m[None, :]) * lax.rsqrt(v + _EPS)[None, :]
                * g_ref[...][None, :] + be_ref[...][None, :], 0.0)
            if has_next:
                hwn = jnp.dot(h, w_ref[...],
                              preferred_element_type=jnp.float32)
                hwn_ref[...] = hwn
                un_ref[...] = hwn * dinv

            @pl.when(i == 0)
            def _():
                psum_ref[...] = jnp.zeros_like(psum_ref)
                pmax_ref[...] = jnp.full_like(pmax_ref, -jnp.inf)
                cnt_ref[...] = jnp.zeros_like(cnt_ref)

            bcol = bc_ref[...]
            gr = lax.broadcasted_iota(jnp.int32, (1, _G), 1)
            maskT = (bcol == gr).astype(jnp.float32)
            psum_ref[...] += lax.dot_general(
                maskT, h, (((0,), (0,)), ((), ())),
                preferred_element_type=jnp.float32)
            cnt_ref[...] += lax.dot_general(
                maskT, jnp.ones((_R, 1), jnp.float32),
                (((0,), (0,)), ((), ())),
                preferred_element_type=jnp.float32)

            glo = bc_ref[0, 0]
            ghi = bc_ref[_R - 1, 0]

            def gbody(gid, carry):
                rm = bcol == gid
                hm = jnp.where(rm, h, -jnp.inf)
                gmax = jnp.max(hm, axis=0)[None, :]
                cur = pmax_ref[pl.ds(gid, 1), :]
                pmax_ref[pl.ds(gid, 1), :] = jnp.maximum(cur, gmax)
                return carry

            lax.fori_loop(glo, ghi + 1, gbody, 0)

    in_specs = [
        pl.BlockSpec((_NC, _R, _LANE), lambda p, i: (0, i, 0)),
        pl.BlockSpec((_R, _D), lambda p, i: (i, 0)),
        pl.BlockSpec((_R, 1), lambda p, i: (i, 0)),
        pl.BlockSpec((_D,), lambda p, i: (0,)),
        pl.BlockSpec((_D,), lambda p, i: (0,)),
        pl.BlockSpec((_D,), lambda p, i: (0,)),
        pl.BlockSpec((_R, 1), lambda p, i: (i, 0)),
    ]
    out_specs = [
        pl.BlockSpec((_G, _D), lambda p, i: (0, 0)),
        pl.BlockSpec((_G, _D), lambda p, i: (0, 0)),
        pl.BlockSpec((_G, 1), lambda p, i: (0, 0)),
    ]
    out_shape = [
        jax.ShapeDtypeStruct((_G, _D), jnp.float32),
        jax.ShapeDtypeStruct((_G, _D), jnp.float32),
        jax.ShapeDtypeStruct((_G, 1), jnp.float32),
    ]
    args = [sagg, hw, dinv, b, g, be, batch_col]
    if has_next:
        in_specs.append(pl.BlockSpec((_D, _D), lambda p, i: (0, 0)))
        out_specs = [
            pl.BlockSpec((_R, _D), lambda p, i: (i, 0)),
            pl.BlockSpec((_R, _D), lambda p, i: (i, 0)),
        ] + out_specs
        out_shape = [
            jax.ShapeDtypeStruct((_N, _D), jnp.float32),
            jax.ShapeDtypeStruct((_N, _D), jnp.float32),
        ] + out_shape
        args.append(W_next)

    return pl.pallas_call(
        body,
        grid=(2, _NB),
        in_specs=in_specs,
        out_specs=out_specs,
        out_shape=out_shape,
        scratch_shapes=[pltpu.VMEM((2, _D), jnp.float32)],
    )(*args)


def _tc_head(psums, pmaxs, cnt, lw1, lb1, lw2, lb2, lw3, lb3):
    def body(ps_ref, pm_ref, cnt_ref, lw1_ref, lb1_ref, lw2_ref, lb2_ref,
             lw3_ref, lb3_ref, o_ref):
        mx = pm_ref[0] + pm_ref[1] + pm_ref[2] + pm_ref[3]
        sm = ps_ref[0] + ps_ref[1] + ps_ref[2] + ps_ref[3]
        mean = sm / jnp.maximum(cnt_ref[...], 1.0)
        z1 = jnp.maximum(
            jnp.dot(mx, lw1_ref[0:_D, :], preferred_element_type=jnp.float32)
            + jnp.dot(mean, lw1_ref[_D:2 * _D, :],
                      preferred_element_type=jnp.float32)
            + lb1_ref[...][None, :], 0.0)
        z2 = jnp.maximum(
            jnp.dot(z1, lw2_ref[...], preferred_element_type=jnp.float32)
            + lb2_ref[...][None, :], 0.0)
        o_ref[...] = (jnp.dot(z2, lw3_ref[...],
                              preferred_element_type=jnp.float32)
                      + lb3_ref[...][None, :])

    return pl.pallas_call(
        body,
        out_shape=jax.ShapeDtypeStruct((_G, 128), jnp.float32),
    )(psums, pmaxs, cnt, lw1, lb1, lw2, lb2, lw3, lb3)


def kernel(x, edge_index, batch, W1, b1, W2, b2, W3, b3, W4, b4,
           g1, be1, g2, be2, g3, be3, g4, be4,
           lw1, lb1, lw2, lb2, lw3, lb3):
    src = edge_index[0]
    dst = edge_index[1]
    pad = _EPAD - _E
    src_p = jnp.concatenate([src, jnp.zeros((pad,), jnp.int32)])
    dst_p = jnp.concatenate([dst, jnp.full((pad,), _DUMMY, jnp.int32)])
    gidx = jnp.stack([src_p * 2, src_p * 2 + 1]).reshape(
        _NC, _NS, _NCHUNK, _LANE)
    dsts = dst_p.reshape(_NS, _NCHUNK, _LANE)
    zrows = jnp.zeros((_ZROWS, _LANE), jnp.float32)

    degt = _sc_degree(dsts, jnp.zeros((_ZROWS, 16), jnp.float32),
                      jnp.ones((_LANE, 16), jnp.float32))
    deg_col = degt[0, :_N, 0:1]

    hw, u, dinv = _tc_layer1(x, W1, deg_col)
    batch_col = batch[:, None]

    bs = [b1, b2, b3, b4]
    gs = [g1, g2, g3, g4]
    bes = [be1, be2, be3, be4]
    Ws = [W2, W3, W4, None]
    psums, pmaxs, cnt = [], [], None
    for l in range(4):
        sagg = _sc_segment_sum(u.reshape(2 * _N, _LANE), gidx, dsts,
                               zrows)[:, :_N, :]
        if Ws[l] is not None:
            hw_n, u_n, ps, pm, c0 = _tc_main(sagg, hw, dinv, bs[l],
                                             gs[l], bes[l], batch_col, Ws[l])
            hw, u = hw_n, u_n
        else:
            ps, pm, c0 = _tc_main(sagg, hw, dinv, bs[l], gs[l], bes[l],
                                  batch_col, None)
        if cnt is None:
            cnt = c0
        psums.append(ps)
        pmaxs.append(pm)

    return _tc_head(jnp.stack(psums), jnp.stack(pmaxs), cnt,
                    lw1, lb1, lw2, lb2, lw3, lb3)


# speedup vs baseline: 1.0222x; 1.0222x over previous
import functools

import jax
import jax.numpy as jnp
from jax import lax
from jax.experimental import pallas as pl
from jax.experimental.pallas import tpu as pltpu
from jax.experimental.pallas import tpu_sc as plsc

_N = 10000
_E = 160000
_D = 256
_G = 64
_NC = 2
_NS = 16
_LANE = 128
_EPW = 10240
_EPAD = _NS * _EPW
_NCHUNK = _EPW // _LANE
_ACCROWS = 10240
_DUMMY = _N
_ZROWS = _ACCROWS // _NS
_OROWS = _ACCROWS // _NS
_R = 400
_NB = _N // _R
_EPS = 1e-5


def _sc_degree(dsts, zrows16, ones16):
    mesh = plsc.VectorSubcoreMesh(core_axis_name="c", subcore_axis_name="s")

    @functools.partial(
        pl.kernel,
        out_type=jax.ShapeDtypeStruct((_NC, _ACCROWS, 16), jnp.float32),
        mesh=mesh,
        scratch_types=[
            pltpu.VMEM((_NCHUNK, _LANE), jnp.int32),
            pltpu.VMEM((_LANE, 16), jnp.float32),
            pltpu.VMEM_SHARED((_ACCROWS, 16), jnp.float32),
        ],
    )
    def k(dst_hbm, z_hbm, o_hbm, out_hbm, dst_v, ones_v, acc):
        c = lax.axis_index("c")
        s = lax.axis_index("s")
        pltpu.sync_copy(dst_hbm.at[s], dst_v)
        pltpu.sync_copy(o_hbm, ones_v)
        pltpu.sync_copy(z_hbm, acc.at[pl.ds(s * _ZROWS, _ZROWS)])
        plsc.subcore_barrier()

        def chunk(j, carry):
            pltpu.sync_copy(ones_v, acc.at[dst_v.at[j]], add=True)
            return carry

        lax.fori_loop(0, _NCHUNK, chunk, 0)
        plsc.subcore_barrier()
        pltpu.sync_copy(acc.at[pl.ds(s * _OROWS, _OROWS)],
                        out_hbm.at[c, pl.ds(s * _OROWS, _OROWS)])

    return k(dsts, zrows16, ones16)


def _sc_segment_sum(u2, gidx, dsts, zrows):
    mesh = plsc.VectorSubcoreMesh(core_axis_name="c", subcore_axis_name="s")

    @functools.partial(
        pl.kernel,
        out_type=jax.ShapeDtypeStruct((_NC, _ACCROWS, _LANE), jnp.float32),
        mesh=mesh,
        scratch_types=[
            pltpu.VMEM((_NCHUNK, _LANE), jnp.int32),
            pltpu.VMEM((_NCHUNK, _LANE), jnp.int32),
            pltpu.VMEM((_LANE, _LANE), jnp.float32),
            pltpu.VMEM_SHARED((_ACCROWS, _LANE), jnp.float32),
            pltpu.SemaphoreType.DMA,
        ],
    )
    def k(u_hbm, gidx_hbm, dst_hbm, z_hbm, out_hbm, idx_v, dst_v, rows_v,
          acc, sem):
        c = lax.axis_index("c")
        s = lax.axis_index("s")
        pltpu.sync_copy(gidx_hbm.at[c, s], idx_v)
        pltpu.sync_copy(dst_hbm.at[s], dst_v)
        pltpu.sync_copy(z_hbm, acc.at[pl.ds(s * _ZROWS, _ZROWS)])
        plsc.subcore_barrier()

        def chunk(j, carry):
            pltpu.async_copy(u_hbm.at[idx_v.at[j]], rows_v, sem).wait()
            pltpu.sync_copy(rows_v, acc.at[dst_v.at[j]], add=True)
            return carry

        lax.fori_loop(0, _NCHUNK, chunk, 0)
        plsc.subcore_barrier()
        pltpu.sync_copy(acc.at[pl.ds(s * _OROWS, _OROWS)],
                        out_hbm.at[c, pl.ds(s * _OROWS, _OROWS)])

    return k(u2, gidx, dsts, zrows)


def _tc_layer1(x, W, deg_col):
    def body(x_ref, w_ref, deg_ref, hw_ref, u_ref, dinv_ref):
        hw = jnp.dot(x_ref[...], w_ref[...],
                     preferred_element_type=jnp.float32)
        dinv = lax.rsqrt(deg_ref[...] + 1.0)
        hw_ref[...] = hw
        u_ref[...] = hw * dinv
        dinv_ref[...] = dinv

    return pl.pallas_call(
        body,
        grid=(_NB,),
        in_specs=[
            pl.BlockSpec((_R, _D), lambda i: (i, 0)),
            pl.BlockSpec((_D, _D), lambda i: (0, 0)),
            pl.BlockSpec((_R, 1), lambda i: (i, 0)),
        ],
        out_specs=[
            pl.BlockSpec((_R, _D), lambda i: (i, 0)),
            pl.BlockSpec((_R, _D), lambda i: (i, 0)),
            pl.BlockSpec((_R, 1), lambda i: (i, 0)),
        ],
        out_shape=[
            jax.ShapeDtypeStruct((_N, _D), jnp.float32),
            jax.ShapeDtypeStruct((_N, _D), jnp.float32),
            jax.ShapeDtypeStruct((_N, 1), jnp.float32),
        ],
    )(x, W, deg_col)


def _pre_block(s_ref, hw_ref, dinv, b_ref):
    sb = jnp.concatenate([s_ref[0], s_ref[1]], axis=1)
    return dinv * sb + (dinv * dinv) * hw_ref[...] + b_ref[...][None, :]


def _tc_stats(sagg, hw, dinv, b):
    def body(s_ref, hw_ref, dinv_ref, b_ref, st_ref):
        i = pl.program_id(0)
        pre = _pre_block(s_ref, hw_ref, dinv_ref[...], b_ref)

        @pl.when(i == 0)
        def _():
            st_ref[...] = jnp.zeros_like(st_ref)

        st_ref[...] += jnp.concatenate(
            [jnp.sum(pre, axis=0)[None, :],
             jnp.sum(pre * pre, axis=0)[None, :]], axis=0)

    return pl.pallas_call(
        body,
        grid=(_NB,),
        in_specs=[
            pl.BlockSpec((_NC, _R, _LANE), lambda i: (0, i, 0)),
            pl.BlockSpec((_R, _D), lambda i: (i, 0)),
            pl.BlockSpec((_R, 1), lambda i: (i, 0)),
            pl.BlockSpec((_D,), lambda i: (0,)),
        ],
        out_specs=pl.BlockSpec((2, _D), lambda i: (0, 0)),
        out_shape=jax.ShapeDtypeStruct((2, _D), jnp.float32),
    )(sagg, hw, dinv, b)


def _tc_main(sagg, hw, dinv, st, b, g, be, batch_col, W_next):
    has_next = W_next is not None

    def body(s_ref, hw_ref, dinv_ref, st_ref, b_ref, g_ref, be_ref, bc_ref,
             *rest):
        if has_next:
            w_ref, hwn_ref, un_ref, psum_ref, pmax_ref, cnt_ref = rest
        else:
            psum_ref, pmax_ref, cnt_ref = rest
        i = pl.program_id(0)
        dinv = dinv_ref[...]
        pre = _pre_block(s_ref, hw_ref, dinv, b_ref)
        m = st_ref[0, :] * (1.0 / _N)
        v = st_ref[1, :] * (1.0 / _N) - m * m
        h = jnp.maximum(
            (pre - m[None, :]) * lax.rsqrt(v + _EPS)[None, :]
            * g_ref[...][None, :] + be_ref[...][None, :], 0.0)
        if has_next:
            hwn = jnp.dot(h, w_ref[...], preferred_element_type=jnp.float32)
            hwn_ref[...] = hwn
            un_ref[...] = hwn * dinv

        @pl.when(i == 0)
        def _():
            psum_ref[...] = jnp.zeros_like(psum_ref)
            pmax_ref[...] = jnp.full_like(pmax_ref, -jnp.inf)
            cnt_ref[...] = jnp.zeros_like(cnt_ref)

        bcol = bc_ref[...]
        gr = lax.broadcasted_iota(jnp.int32, (1, _G), 1)
        maskT = (bcol == gr).astype(jnp.float32)
        psum_ref[...] += lax.dot_general(
            maskT, h, (((0,), (0,)), ((), ())),
            preferred_element_type=jnp.float32)
        cnt_ref[...] += lax.dot_general(
            maskT, jnp.ones((_R, 1), jnp.float32), (((0,), (0,)), ((), ())),
            preferred_element_type=jnp.float32)

        glo = bc_ref[0, 0]
        ghi = bc_ref[_R - 1, 0]

        def gbody(gid, carry):
            rm = bcol == gid
            hm = jnp.where(rm, h, -jnp.inf)
            gmax = jnp.max(hm, axis=0)[None, :]
            cur = pmax_ref[pl.ds(gid, 1), :]
            pmax_ref[pl.ds(gid, 1), :] = jnp.maximum(cur, gmax)
            return carry

        lax.fori_loop(glo, ghi + 1, gbody, 0)

    in_specs = [
        pl.BlockSpec((_NC, _R, _LANE), lambda i: (0, i, 0)),
        pl.BlockSpec((_R, _D), lambda i: (i, 0)),
        pl.BlockSpec((_R, 1), lambda i: (i, 0)),
        pl.BlockSpec((2, _D), lambda i: (0, 0)),
        pl.BlockSpec((_D,), lambda i: (0,)),
        pl.BlockSpec((_D,), lambda i: (0,)),
        pl.BlockSpec((_D,), lambda i: (0,)),
        pl.BlockSpec((_R, 1), lambda i: (i, 0)),
    ]
    out_specs = [
        pl.BlockSpec((_G, _D), lambda i: (0, 0)),
        pl.BlockSpec((_G, _D), lambda i: (0, 0)),
        pl.BlockSpec((_G, 1), lambda i: (0, 0)),
    ]
    out_shape = [
        jax.ShapeDtypeStruct((_G, _D), jnp.float32),
        jax.ShapeDtypeStruct((_G, _D), jnp.float32),
        jax.ShapeDtypeStruct((_G, 1), jnp.float32),
    ]
    args = [sagg, hw, dinv, st, b, g, be, batch_col]
    if has_next:
        in_specs.append(pl.BlockSpec((_D, _D), lambda i: (0, 0)))
        out_specs = [
            pl.BlockSpec((_R, _D), lambda i: (i, 0)),
            pl.BlockSpec((_R, _D), lambda i: (i, 0)),
        ] + out_specs
        out_shape = [
            jax.ShapeDtypeStruct((_N, _D), jnp.float32),
            jax.ShapeDtypeStruct((_N, _D), jnp.float32),
        ] + out_shape
        args.append(W_next)

    return pl.pallas_call(
        body,
        grid=(_NB,),
        in_specs=in_specs,
        out_specs=out_specs,
        out_shape=out_shape,
    )(*args)


def _tc_head(psums, pmaxs, cnt, lw1, lb1, lw2, lb2, lw3, lb3):
    def body(ps_ref, pm_ref, cnt_ref, lw1_ref, lb1_ref, lw2_ref, lb2_ref,
             lw3_ref, lb3_ref, o_ref):
        mx = pm_ref[0] + pm_ref[1] + pm_ref[2] + pm_ref[3]
        sm = ps_ref[0] + ps_ref[1] + ps_ref[2] + ps_ref[3]
        mean = sm / jnp.maximum(cnt_ref[...], 1.0)
        z1 = jnp.maximum(
            jnp.dot(mx, lw1_ref[0:_D, :], preferred_element_type=jnp.float32)
            + jnp.dot(mean, lw1_ref[_D:2 * _D, :],
                      preferred_element_type=jnp.float32)
            + lb1_ref[...][None, :], 0.0)
        z2 = jnp.maximum(
            jnp.dot(z1, lw2_ref[...], preferred_element_type=jnp.float32)
            + lb2_ref[...][None, :], 0.0)
        o_ref[...] = (jnp.dot(z2, lw3_ref[...],
                              preferred_element_type=jnp.float32)
                      + lb3_ref[...][None, :])

    return pl.pallas_call(
        body,
        out_shape=jax.ShapeDtypeStruct((_G, 128), jnp.float32),
    )(psums, pmaxs, cnt, lw1, lb1, lw2, lb2, lw3, lb3)


def kernel(x, edge_index, batch, W1, b1, W2, b2, W3, b3, W4, b4,
           g1, be1, g2, be2, g3, be3, g4, be4,
           lw1, lb1, lw2, lb2, lw3, lb3):
    src = edge_index[0]
    dst = edge_index[1]
    pad = _EPAD - _E
    src_p = jnp.concatenate([src, jnp.zeros((pad,), jnp.int32)])
    dst_p = jnp.concatenate([dst, jnp.full((pad,), _DUMMY, jnp.int32)])
    gidx = jnp.stack([src_p * 2, src_p * 2 + 1]).reshape(
        _NC, _NS, _NCHUNK, _LANE)
    dsts = dst_p.reshape(_NS, _NCHUNK, _LANE)
    zrows = jnp.zeros((_ZROWS, _LANE), jnp.float32)

    degt = _sc_degree(dsts, jnp.zeros((_ZROWS, 16), jnp.float32),
                      jnp.ones((_LANE, 16), jnp.float32))
    deg_col = degt[0, :_N, 0:1]

    hw, u, dinv = _tc_layer1(x, W1, deg_col)
    batch_col = batch[:, None]

    bs = [b1, b2, b3, b4]
    gs = [g1, g2, g3, g4]
    bes = [be1, be2, be3, be4]
    Ws = [W2, W3, W4, None]
    psums, pmaxs, cnt = [], [], None
    for l in range(4):
        sagg = _sc_segment_sum(u.reshape(2 * _N, _LANE), gidx, dsts, zrows)
        st = _tc_stats(sagg, hw, dinv, bs[l])
        if Ws[l] is not None:
            hw_n, u_n, ps, pm, c0 = _tc_main(sagg, hw, dinv, st, bs[l],
                                             gs[l], bes[l], batch_col, Ws[l])
            hw, u = hw_n, u_n
        else:
            ps, pm, c0 = _tc_main(sagg, hw, dinv, st, bs[l], gs[l], bes[l],
                                  batch_col, None)
        if cnt is None:
            cnt = c0
        psums.append(ps)
        pmaxs.append(pm)

    return _tc_head(jnp.stack(psums), jnp.stack(pmaxs), cnt,
                    lw1, lb1, lw2, lb2, lw3, lb3)
